# baseline (speedup 1.0000x reference)
import functools

import jax
import jax.numpy as jnp
from jax import lax
from jax.experimental import pallas as pl
from jax.experimental.pallas import tpu as pltpu
from jax.experimental.pallas import tpu_sc as plsc

_NC = 81
_N = 20000
_K = 64
_NP = 20480
_CHUNK = 2048
_GRID = _NP // _CHUNK
_ROWS = _NP // 128
_MIN32 = -2147483648


def _rowstats_kernel(st_ref, lab_ref, pos_ref, neg_ref, cfg_ref, cbg_ref):
    s = st_ref[...].T
    lab = lab_ref[...]
    gcol = (pl.program_id(0) * _CHUNK
            + lax.broadcasted_iota(jnp.int32, (1, _CHUNK), 1))
    valid = gcol < _N

    m80 = jnp.max(s[:_NC - 1, :], axis=0, keepdims=True)
    m_ex = jnp.maximum(m80, s[_NC:_NC + 1, :])
    metric = -m_ex
    b = lax.bitcast_convert_type(metric, jnp.int32)
    key = jnp.where(b < 0, b ^ jnp.int32(0x7FFFFFFF), b)
    fg = (lab != _NC) & valid
    bg = (lab == _NC) & valid
    masked = jnp.int32(_MIN32)
    pos_ref[...] = jnp.where(fg, key, masked)
    neg_ref[...] = jnp.where(bg, key, masked)

    m_all = jnp.maximum(m_ex, s[_NC - 1:_NC, :])
    e = jnp.exp(s - m_all)
    ssum = jnp.sum(e, axis=0, keepdims=True)
    row = lax.broadcasted_iota(jnp.int32, s.shape, 0)
    e_lab = jnp.sum(jnp.where(row == lab, e, 0.0), axis=0, keepdims=True)
    gt = e_lab / ssum
    t = gt * (1.0 - gt)
    denomlog = m_all + jnp.log(ssum - e_lab)
    s79 = s[_NC - 2:_NC - 1, :]
    s80 = s[_NC - 1:_NC, :]
    s81 = s[_NC:_NC + 1, :]
    c_fg = jnp.where(lab <= _NC - 2, s80, s79)
    c_bg = jnp.where(lab <= _NC - 1, s81, s80)
    cfg_ref[...] = t * (c_fg - denomlog)
    cbg_ref[...] = t * (c_bg - denomlog)


_NSUB = 16
_PER = _NP // _NSUB
_NCH = _PER // 16


def _sc_select_body(kpu_hbm, knu_hbm, cf_hbm, cb_hbm, out_hbm,
                    kup, kun, cfv, cbv, hist, allh, allf, stage,
                    sharedh, sharedf):
    cid = lax.axis_index("c")
    wid = lax.axis_index("s")
    base = wid * _PER
    iota16 = lax.iota(jnp.int32, 16)
    ones16 = jnp.ones((16,), jnp.int32)
    zeros16 = jnp.zeros((16,), jnp.int32)
    k = jnp.int32(_K)
    imax = jnp.int32(0x7FFFFFFF)

    pltpu.sync_copy(kpu_hbm.at[pl.ds(base, _PER)], kup)
    pltpu.sync_copy(knu_hbm.at[pl.ds(base, _PER)], kun)
    pltpu.sync_copy(cf_hbm.at[pl.ds(base, _PER)], cfv)
    pltpu.sync_copy(cb_hbm.at[pl.ds(base, _PER)], cbv)

    sgn = jnp.uint32(0x80000000)

    def conv(c, _):
        kup[pl.ds(c * 16, 16)] = kup[pl.ds(c * 16, 16)] ^ sgn
        kun[pl.ds(c * 16, 16)] = kun[pl.ds(c * 16, 16)] ^ sgn
        return 0

    lax.fori_loop(0, _NCH, conv, 0, unroll=8)

    def zero_hist():
        for i in range(32):
            hist[pl.ds(i * 16, 16)] = zeros16

    def exchange(reg):
        pltpu.sync_copy(hist, sharedh.at[pl.ds(reg + wid * 512, 512)])
        plsc.subcore_barrier()
        pltpu.sync_copy(sharedh.at[pl.ds(reg, 8192)], allh)

        def srow(w, acc):
            return tuple(acc[b] + allh[pl.ds(w * 512 + b * 16, 16)]
                         for b in range(32))

        return lax.fori_loop(0, _NSUB, srow, (zeros16,) * 32)

    def pick_desc(blocks, kk):
        bs = [jnp.sum(blocks[b]) for b in range(16)]
        suf = jnp.int32(0)
        suf_after = [None] * 16
        for b in range(15, -1, -1):
            suf_after[b] = suf
            suf = suf + bs[b]
        gcnt = jnp.int32(0)
        rcbs = []
        for b in range(16):
            rcb = jnp.flip(jnp.cumsum(jnp.flip(blocks[b], 0)), 0) + suf_after[b]
            rcbs.append(rcb)
            gcnt = gcnt + jnp.sum((rcb >= kk).astype(jnp.int32))
        g = gcnt - 1
        rc_g = jnp.int32(0)
        rc_g1 = jnp.int32(0)
        for b in range(16):
            lid = iota16 + 16 * b
            rc_g = rc_g + jnp.sum(jnp.where(lid == g, rcbs[b], 0))
            rc_g1 = rc_g1 + jnp.sum(jnp.where(lid == g + 1, rcbs[b], 0))
        return g, rc_g1, rc_g - rc_g1

    def pick_asc(blocks, kk):
        bs = [jnp.sum(blocks[b]) for b in range(16)]
        pre = jnp.int32(0)
        pre_before = []
        for b in range(16):
            pre_before.append(pre)
            pre = pre + bs[b]
        gcnt = jnp.int32(0)
        fcbs = []
        for b in range(16):
            fcb = jnp.cumsum(blocks[b]) + pre_before[b]
            fcbs.append(fcb)
            gcnt = gcnt + jnp.sum((fcb < kk).astype(jnp.int32))
        prev = jnp.int32(0)
        for b in range(16):
            lid = iota16 + 16 * b
            prev = prev + jnp.sum(jnp.where(lid == gcnt - 1, fcbs[b], 0))
        return gcnt, prev

    def val_iter(t, carry):
        vp, kp1, vn, kn1, _, _ = carry
        sh = jnp.uint32(24) - jnp.uint32(8) * t.astype(jnp.uint32)
        reg = (t & 1) * 8192
        zero_hist()

        def chunk(c, _):
            xp = kup[pl.ds(c * 16, 16)]
            xn = kun[pl.ds(c * 16, 16)]
            vmp = (xp >> sh >> 8) == (vp >> sh >> 8)
            vmn = (xn >> sh >> 8) == (vn >> sh >> 8)
            nibp = ((xp >> sh) & jnp.uint32(255)).astype(jnp.int32)
            nibn = ((xn >> sh) & jnp.uint32(255)).astype(jnp.int32) + 256
            plsc.addupdate_scatter(hist, [nibp], ones16, mask=vmp)
            plsc.addupdate_scatter(hist, [nibn], ones16, mask=vmn)
            return 0

        lax.fori_loop(0, _NCH, chunk, 0, unroll=8)
        blocks = exchange(reg)
        gp, nxtp, tiep = pick_desc(blocks[:16], kp1)
        gn, nxtn, tien = pick_desc(blocks[16:], kn1)
        kp1 = kp1 - nxtp
        kn1 = kn1 - nxtn
        vp = vp | (gp.astype(jnp.uint32) << sh)
        vn = vn | (gn.astype(jnp.uint32) << sh)
        return vp, kp1, vn, kn1, tiep, tien

    vp, kp1, vn, kn1, tiecp, tiecn = lax.fori_loop(
        0, 4, val_iter,
        (jnp.uint32(0), k, jnp.uint32(0), k, jnp.int32(0), jnp.int32(0)))

    skip = (tiecp == kp1) & (tiecn == kn1)
    trip = jnp.where(skip, 0, 2)
    init_ip = jnp.where(skip, imax, jnp.int32(0))

    def idx_iter(t, carry):
        ipx, k2p, inx, k2n = carry
        sh = 8 - 8 * t
        reg = (t & 1) * 8192
        zero_hist()

        def chunk(c, _):
            xp = kup[pl.ds(c * 16, 16)]
            xn = kun[pl.ds(c * 16, 16)]
            gidx = base + c * 16 + iota16
            vmp = (xp == vp) & ((gidx >> sh >> 8) == (ipx >> sh >> 8))
            vmn = (xn == vn) & ((gidx >> sh >> 8) == (inx >> sh >> 8))
            nib = (gidx >> sh) & 255
            plsc.addupdate_scatter(hist, [nib], ones16, mask=vmp)
            plsc.addupdate_scatter(hist, [nib + 256], ones16, mask=vmn)
            return 0

        lax.fori_loop(0, _NCH, chunk, 0, unroll=8)
        blocks = exchange(reg)
        gp, prevp = pick_asc(blocks[:16], k2p)
        gn, prevn = pick_asc(blocks[16:], k2n)
        k2p = k2p - prevp
        k2n = k2n - prevn
        ipx = ipx | (gp << sh)
        inx = inx | (gn << sh)
        return ipx, k2p, inx, k2n

    tpx, _, tnx, _ = lax.fori_loop(
        0, trip, idx_iter, (init_ip, kp1, init_ip, kn1))

    def psum(c, acc):
        xp = kup[pl.ds(c * 16, 16)]
        xn = kun[pl.ds(c * 16, 16)]
        gidx = base + c * 16 + iota16
        sp = (xp > vp) | ((xp == vp) & (gidx <= tpx))
        sn = (xn > vn) | ((xn == vn) & (gidx <= tnx))
        v = jnp.where(sp, cfv[pl.ds(c * 16, 16)], 0.0)
        v = v + jnp.where(sn, cbv[pl.ds(c * 16, 16)], 0.0)
        return acc + v

    part = lax.fori_loop(0, _NCH, psum, jnp.zeros((16,), jnp.float32), unroll=8)
    stage[...] = part
    pltpu.sync_copy(stage, sharedf.at[pl.ds(wid * 16, 16)])
    plsc.subcore_barrier()

    @pl.when((cid == 0) & (wid == 0))
    def _():
        pltpu.sync_copy(sharedf, allf)

        def fs(w, acc):
            return acc + allf[pl.ds(w * 16, 16)]

        tot = lax.fori_loop(0, _NSUB, fs, jnp.zeros((16,), jnp.float32))
        stage[...] = (jnp.broadcast_to(jnp.sum(tot), (16,))
                      * jnp.float32(-1.0 / (2 * _K)))
        pltpu.sync_copy(stage, out_hbm)


def _make_sc_select():
    mesh = plsc.VectorSubcoreMesh(core_axis_name="c", subcore_axis_name="s")
    return pl.kernel(
        _sc_select_body,
        out_type=jax.ShapeDtypeStruct((16,), jnp.float32),
        mesh=mesh,
        compiler_params=pltpu.CompilerParams(needs_layout_passes=False),
        scratch_types=[
            pltpu.VMEM((_PER,), jnp.uint32),
            pltpu.VMEM((_PER,), jnp.uint32),
            pltpu.VMEM((_PER,), jnp.float32),
            pltpu.VMEM((_PER,), jnp.float32),
            pltpu.VMEM((512,), jnp.int32),
            pltpu.VMEM((8192,), jnp.int32),
            pltpu.VMEM((256,), jnp.float32),
            pltpu.VMEM((16,), jnp.float32),
            pltpu.VMEM_SHARED((16384,), jnp.int32),
            pltpu.VMEM_SHARED((256,), jnp.float32),
        ],
    )


@jax.jit
def kernel(scores, labels, un_id, weight, bias):
    lab2 = jnp.pad(labels.reshape(1, _N), ((0, 0), (0, _NP - _N)))
    pos, neg, cfg, cbg = pl.pallas_call(
        _rowstats_kernel,
        grid=(_GRID,),
        in_specs=[
            pl.BlockSpec((_CHUNK, _NC + 1), lambda i: (i, 0)),
            pl.BlockSpec((1, _CHUNK), lambda i: (0, i)),
        ],
        out_specs=[pl.BlockSpec((1, _CHUNK), lambda i: (0, i))] * 4,
        out_shape=[jax.ShapeDtypeStruct((1, _NP), jnp.int32)] * 2
        + [jax.ShapeDtypeStruct((1, _NP), jnp.float32)] * 2,
    )(scores, lab2)

    kpu = lax.bitcast_convert_type(pos.reshape(_NP), jnp.uint32)
    knu = lax.bitcast_convert_type(neg.reshape(_NP), jnp.uint32)
    out = _make_sc_select()(kpu, knu, cfg.reshape(_NP), cbg.reshape(_NP))
    return out[0]


# speedup vs baseline: 1.0810x; 1.0810x over previous
import jax
import jax.numpy as jnp
from jax import lax
from jax.experimental import pallas as pl
from jax.experimental.pallas import tpu as pltpu
from jax.experimental.pallas import tpu_sc as plsc

_NC = 81
_N = 20000
_K = 64
_NP = 20480
_CHUNK = 2048
_GRID = _NP // _CHUNK
_ROWS = _NP // 128
_MIN32 = -2147483648


def _rowstats_kernel(st_ref, lab_ref, pos_ref, neg_ref, cfg_ref, cbg_ref):
    s = st_ref[...]
    lab = lab_ref[...]
    gcol = (pl.program_id(0) * _CHUNK
            + lax.broadcasted_iota(jnp.int32, (1, _CHUNK), 1))
    valid = gcol < _N

    m80 = jnp.max(s[:_NC - 1, :], axis=0, keepdims=True)
    m_ex = jnp.maximum(m80, s[_NC:_NC + 1, :])
    metric = -m_ex
    b = lax.bitcast_convert_type(metric, jnp.int32)
    key = jnp.where(b < 0, b ^ jnp.int32(0x7FFFFFFF), b)
    fg = (lab != _NC) & valid
    bg = (lab == _NC) & valid
    masked = jnp.int32(_MIN32)
    pos_ref[...] = jnp.where(fg, key, masked)
    neg_ref[...] = jnp.where(bg, key, masked)

    m_all = jnp.maximum(m_ex, s[_NC - 1:_NC, :])
    e = jnp.exp(s - m_all)
    ssum = jnp.sum(e, axis=0, keepdims=True)
    row = lax.broadcasted_iota(jnp.int32, s.shape, 0)
    e_lab = jnp.sum(jnp.where(row == lab, e, 0.0), axis=0, keepdims=True)
    gt = e_lab / ssum
    t = gt * (1.0 - gt)
    denomlog = m_all + jnp.log(ssum - e_lab)
    s79 = s[_NC - 2:_NC - 1, :]
    s80 = s[_NC - 1:_NC, :]
    s81 = s[_NC:_NC + 1, :]
    c_fg = jnp.where(lab <= _NC - 2, s80, s79)
    c_bg = jnp.where(lab <= _NC - 1, s81, s80)
    cfg_ref[...] = t * (c_fg - denomlog)
    cbg_ref[...] = t * (c_bg - denomlog)


_NSUB = 16
_PER = _NP // _NSUB
_NCH = _PER // 16


def _sc_select_body(kpu_hbm, knu_hbm, cf_hbm, cb_hbm, out_hbm,
                    kup, kun, cfv, cbv, hist, allh, allf, stage,
                    sharedh, sharedf):
    cid = lax.axis_index("c")
    wid = lax.axis_index("s")
    base = wid * _PER
    iota16 = lax.iota(jnp.int32, 16)
    ones16 = jnp.ones((16,), jnp.int32)
    zeros16 = jnp.zeros((16,), jnp.int32)
    k = jnp.int32(_K)
    imax = jnp.int32(0x7FFFFFFF)

    pltpu.sync_copy(kpu_hbm.at[pl.ds(base, _PER)], kup)
    pltpu.sync_copy(knu_hbm.at[pl.ds(base, _PER)], kun)
    pltpu.sync_copy(cf_hbm.at[pl.ds(base, _PER)], cfv)
    pltpu.sync_copy(cb_hbm.at[pl.ds(base, _PER)], cbv)

    sgn = jnp.uint32(0x80000000)

    def conv(c, _):
        kup[pl.ds(c * 16, 16)] = kup[pl.ds(c * 16, 16)] ^ sgn
        kun[pl.ds(c * 16, 16)] = kun[pl.ds(c * 16, 16)] ^ sgn
        return 0

    lax.fori_loop(0, _NCH, conv, 0, unroll=8)

    def zero_hist():
        for i in range(32):
            hist[pl.ds(i * 16, 16)] = zeros16

    def exchange(reg):
        pltpu.sync_copy(hist, sharedh.at[pl.ds(reg + wid * 512, 512)])
        plsc.subcore_barrier()
        pltpu.sync_copy(sharedh.at[pl.ds(reg, 8192)], allh)

        def srow(w, acc):
            return tuple(acc[b] + allh[pl.ds(w * 512 + b * 16, 16)]
                         for b in range(32))

        return lax.fori_loop(0, _NSUB, srow, (zeros16,) * 32)

    def pick_desc(blocks, kk):
        bs = [jnp.sum(blocks[b]) for b in range(16)]
        suf = jnp.int32(0)
        suf_after = [None] * 16
        for b in range(15, -1, -1):
            suf_after[b] = suf
            suf = suf + bs[b]
        gcnt = jnp.int32(0)
        rcbs = []
        for b in range(16):
            rcb = jnp.flip(jnp.cumsum(jnp.flip(blocks[b], 0)), 0) + suf_after[b]
            rcbs.append(rcb)
            gcnt = gcnt + jnp.sum((rcb >= kk).astype(jnp.int32))
        g = gcnt - 1
        rc_g = jnp.int32(0)
        rc_g1 = jnp.int32(0)
        for b in range(16):
            lid = iota16 + 16 * b
            rc_g = rc_g + jnp.sum(jnp.where(lid == g, rcbs[b], 0))
            rc_g1 = rc_g1 + jnp.sum(jnp.where(lid == g + 1, rcbs[b], 0))
        return g, rc_g1, rc_g - rc_g1

    def pick_asc(blocks, kk):
        bs = [jnp.sum(blocks[b]) for b in range(16)]
        pre = jnp.int32(0)
        pre_before = []
        for b in range(16):
            pre_before.append(pre)
            pre = pre + bs[b]
        gcnt = jnp.int32(0)
        fcbs = []
        for b in range(16):
            fcb = jnp.cumsum(blocks[b]) + pre_before[b]
            fcbs.append(fcb)
            gcnt = gcnt + jnp.sum((fcb < kk).astype(jnp.int32))
        prev = jnp.int32(0)
        for b in range(16):
            lid = iota16 + 16 * b
            prev = prev + jnp.sum(jnp.where(lid == gcnt - 1, fcbs[b], 0))
        return gcnt, prev

    def val_iter(t, carry):
        vp, kp1, vn, kn1, _, _ = carry
        sh = jnp.uint32(24) - jnp.uint32(8) * t.astype(jnp.uint32)
        reg = (t & 1) * 8192
        zero_hist()

        def chunk(c, _):
            xp = kup[pl.ds(c * 16, 16)]
            xn = kun[pl.ds(c * 16, 16)]
            vmp = (xp >> sh >> 8) == (vp >> sh >> 8)
            vmn = (xn >> sh >> 8) == (vn >> sh >> 8)
            nibp = ((xp >> sh) & jnp.uint32(255)).astype(jnp.int32)
            nibn = ((xn >> sh) & jnp.uint32(255)).astype(jnp.int32) + 256
            plsc.addupdate_scatter(hist, [nibp], ones16, mask=vmp)
            plsc.addupdate_scatter(hist, [nibn], ones16, mask=vmn)
            return 0

        lax.fori_loop(0, _NCH, chunk, 0, unroll=8)
        blocks = exchange(reg)
        gp, nxtp, tiep = pick_desc(blocks[:16], kp1)
        gn, nxtn, tien = pick_desc(blocks[16:], kn1)
        kp1 = kp1 - nxtp
        kn1 = kn1 - nxtn
        vp = vp | (gp.astype(jnp.uint32) << sh)
        vn = vn | (gn.astype(jnp.uint32) << sh)
        return vp, kp1, vn, kn1, tiep, tien

    vp, kp1, vn, kn1, tiecp, tiecn = lax.fori_loop(
        0, 4, val_iter,
        (jnp.uint32(0), k, jnp.uint32(0), k, jnp.int32(0), jnp.int32(0)))

    skip = (tiecp == kp1) & (tiecn == kn1)
    trip = jnp.where(skip, 0, 2)
    init_ip = jnp.where(skip, imax, jnp.int32(0))

    def idx_iter(t, carry):
        ipx, k2p, inx, k2n = carry
        sh = 8 - 8 * t
        reg = (t & 1) * 8192
        zero_hist()

        def chunk(c, _):
            xp = kup[pl.ds(c * 16, 16)]
            xn = kun[pl.ds(c * 16, 16)]
            gidx = base + c * 16 + iota16
            vmp = (xp == vp) & ((gidx >> sh >> 8) == (ipx >> sh >> 8))
            vmn = (xn == vn) & ((gidx >> sh >> 8) == (inx >> sh >> 8))
            nib = (gidx >> sh) & 255
            plsc.addupdate_scatter(hist, [nib], ones16, mask=vmp)
            plsc.addupdate_scatter(hist, [nib + 256], ones16, mask=vmn)
            return 0

        lax.fori_loop(0, _NCH, chunk, 0, unroll=8)
        blocks = exchange(reg)
        gp, prevp = pick_asc(blocks[:16], k2p)
        gn, prevn = pick_asc(blocks[16:], k2n)
        k2p = k2p - prevp
        k2n = k2n - prevn
        ipx = ipx | (gp << sh)
        inx = inx | (gn << sh)
        return ipx, k2p, inx, k2n

    tpx, _, tnx, _ = lax.fori_loop(
        0, trip, idx_iter, (init_ip, kp1, init_ip, kn1))

    def psum(c, acc):
        xp = kup[pl.ds(c * 16, 16)]
        xn = kun[pl.ds(c * 16, 16)]
        gidx = base + c * 16 + iota16
        sp = (xp > vp) | ((xp == vp) & (gidx <= tpx))
        sn = (xn > vn) | ((xn == vn) & (gidx <= tnx))
        v = jnp.where(sp, cfv[pl.ds(c * 16, 16)], 0.0)
        v = v + jnp.where(sn, cbv[pl.ds(c * 16, 16)], 0.0)
        return acc + v

    part = lax.fori_loop(0, _NCH, psum, jnp.zeros((16,), jnp.float32), unroll=8)
    stage[...] = part
    pltpu.sync_copy(stage, sharedf.at[pl.ds(wid * 16, 16)])
    plsc.subcore_barrier()

    @pl.when((cid == 0) & (wid == 0))
    def _():
        pltpu.sync_copy(sharedf, allf)

        def fs(w, acc):
            return acc + allf[pl.ds(w * 16, 16)]

        tot = lax.fori_loop(0, _NSUB, fs, jnp.zeros((16,), jnp.float32))
        stage[...] = (jnp.broadcast_to(jnp.sum(tot), (16,))
                      * jnp.float32(-1.0 / (2 * _K)))
        pltpu.sync_copy(stage, out_hbm)


def _make_sc_select():
    mesh = plsc.VectorSubcoreMesh(core_axis_name="c", subcore_axis_name="s")
    return pl.kernel(
        _sc_select_body,
        out_type=jax.ShapeDtypeStruct((16,), jnp.float32),
        mesh=mesh,
        compiler_params=pltpu.CompilerParams(needs_layout_passes=False),
        scratch_types=[
            pltpu.VMEM((_PER,), jnp.uint32),
            pltpu.VMEM((_PER,), jnp.uint32),
            pltpu.VMEM((_PER,), jnp.float32),
            pltpu.VMEM((_PER,), jnp.float32),
            pltpu.VMEM((512,), jnp.int32),
            pltpu.VMEM((8192,), jnp.int32),
            pltpu.VMEM((256,), jnp.float32),
            pltpu.VMEM((16,), jnp.float32),
            pltpu.VMEM_SHARED((16384,), jnp.int32),
            pltpu.VMEM_SHARED((256,), jnp.float32),
        ],
    )


@jax.jit
def kernel(scores, labels, un_id, weight, bias):
    st = jnp.pad(scores.T, ((0, 0), (0, _NP - _N)))
    lab2 = jnp.pad(labels.reshape(1, _N), ((0, 0), (0, _NP - _N)))
    pos, neg, cfg, cbg = pl.pallas_call(
        _rowstats_kernel,
        grid=(_GRID,),
        in_specs=[
            pl.BlockSpec((_NC + 1, _CHUNK), lambda i: (0, i)),
            pl.BlockSpec((1, _CHUNK), lambda i: (0, i)),
        ],
        out_specs=[pl.BlockSpec((1, _CHUNK), lambda i: (0, i))] * 4,
        out_shape=[jax.ShapeDtypeStruct((1, _NP), jnp.int32)] * 2
        + [jax.ShapeDtypeStruct((1, _NP), jnp.float32)] * 2,
    )(st, lab2)

    kpu = lax.bitcast_convert_type(pos.reshape(_NP), jnp.uint32)
    knu = lax.bitcast_convert_type(neg.reshape(_NP), jnp.uint32)
    out = _make_sc_select()(kpu, knu, cfg.reshape(_NP), cbg.reshape(_NP))
    return out[0]


# speedup vs baseline: 1.1252x; 1.0409x over previous
import jax
import jax.numpy as jnp
from jax import lax
from jax.experimental import pallas as pl
from jax.experimental.pallas import tpu as pltpu
from jax.experimental.pallas import tpu_sc as plsc

_NC = 81
_N = 20000
_K = 64
_NP = 20480
_CHUNK = 4096
_GRID = _NP // _CHUNK
_ROWS = _NP // 128
_MIN32 = -2147483648


def _rowstats_kernel(st_ref, lab_ref, pos_ref, neg_ref, cfg_ref, cbg_ref):
    s = st_ref[...]
    lab = lab_ref[...]
    gcol = (pl.program_id(0) * _CHUNK
            + lax.broadcasted_iota(jnp.int32, (1, _CHUNK), 1))
    valid = gcol < _N

    m80 = jnp.max(s[:_NC - 1, :], axis=0, keepdims=True)
    m_ex = jnp.maximum(m80, s[_NC:_NC + 1, :])
    metric = -m_ex
    b = lax.bitcast_convert_type(metric, jnp.int32)
    key = jnp.where(b < 0, b ^ jnp.int32(0x7FFFFFFF), b)
    fg = (lab != _NC) & valid
    bg = (lab == _NC) & valid
    masked = jnp.int32(_MIN32)
    pos_ref[...] = jnp.where(fg, key, masked)
    neg_ref[...] = jnp.where(bg, key, masked)

    m_all = jnp.maximum(m_ex, s[_NC - 1:_NC, :])
    e = jnp.exp(s - m_all)
    ssum = jnp.sum(e, axis=0, keepdims=True)
    row = lax.broadcasted_iota(jnp.int32, s.shape, 0)
    e_lab = jnp.sum(jnp.where(row == lab, e, 0.0), axis=0, keepdims=True)
    gt = e_lab / ssum
    t = gt * (1.0 - gt)
    denomlog = m_all + jnp.log(ssum - e_lab)
    s79 = s[_NC - 2:_NC - 1, :]
    s80 = s[_NC - 1:_NC, :]
    s81 = s[_NC:_NC + 1, :]
    c_fg = jnp.where(lab <= _NC - 2, s80, s79)
    c_bg = jnp.where(lab <= _NC - 1, s81, s80)
    cfg_ref[...] = t * (c_fg - denomlog)
    cbg_ref[...] = t * (c_bg - denomlog)


_NSUB = 16
_PER = _NP // _NSUB
_NCH = _PER // 16


def _sc_select_body(kpu_hbm, knu_hbm, cf_hbm, cb_hbm, out_hbm,
                    kup, kun, cfv, cbv, hist, allh, allf, stage,
                    sharedh, sharedf):
    cid = lax.axis_index("c")
    wid = lax.axis_index("s")
    base = wid * _PER
    iota16 = lax.iota(jnp.int32, 16)
    ones16 = jnp.ones((16,), jnp.int32)
    zeros16 = jnp.zeros((16,), jnp.int32)
    k = jnp.int32(_K)
    imax = jnp.int32(0x7FFFFFFF)

    pltpu.sync_copy(kpu_hbm.at[pl.ds(base, _PER)], kup)
    pltpu.sync_copy(knu_hbm.at[pl.ds(base, _PER)], kun)
    pltpu.sync_copy(cf_hbm.at[pl.ds(base, _PER)], cfv)
    pltpu.sync_copy(cb_hbm.at[pl.ds(base, _PER)], cbv)

    sgn = jnp.uint32(0x80000000)

    def conv(c, _):
        kup[pl.ds(c * 16, 16)] = kup[pl.ds(c * 16, 16)] ^ sgn
        kun[pl.ds(c * 16, 16)] = kun[pl.ds(c * 16, 16)] ^ sgn
        return 0

    lax.fori_loop(0, _NCH, conv, 0, unroll=8)

    def zero_hist():
        for i in range(32):
            hist[pl.ds(i * 16, 16)] = zeros16

    def exchange(reg):
        pltpu.sync_copy(hist, sharedh.at[pl.ds(reg + wid * 512, 512)])
        plsc.subcore_barrier()
        pltpu.sync_copy(sharedh.at[pl.ds(reg, 8192)], allh)

        def srow(w, acc):
            return tuple(acc[b] + allh[pl.ds(w * 512 + b * 16, 16)]
                         for b in range(32))

        return lax.fori_loop(0, _NSUB, srow, (zeros16,) * 32)

    def pick_desc(blocks, kk):
        bs = [jnp.sum(blocks[b]) for b in range(16)]
        suf = jnp.int32(0)
        suf_after = [None] * 16
        for b in range(15, -1, -1):
            suf_after[b] = suf
            suf = suf + bs[b]
        gcnt = jnp.int32(0)
        rcbs = []
        for b in range(16):
            rcb = jnp.flip(jnp.cumsum(jnp.flip(blocks[b], 0)), 0) + suf_after[b]
            rcbs.append(rcb)
            gcnt = gcnt + jnp.sum((rcb >= kk).astype(jnp.int32))
        g = gcnt - 1
        rc_g = jnp.int32(0)
        rc_g1 = jnp.int32(0)
        for b in range(16):
            lid = iota16 + 16 * b
            rc_g = rc_g + jnp.sum(jnp.where(lid == g, rcbs[b], 0))
            rc_g1 = rc_g1 + jnp.sum(jnp.where(lid == g + 1, rcbs[b], 0))
        return g, rc_g1, rc_g - rc_g1

    def pick_asc(blocks, kk):
        bs = [jnp.sum(blocks[b]) for b in range(16)]
        pre = jnp.int32(0)
        pre_before = []
        for b in range(16):
            pre_before.append(pre)
            pre = pre + bs[b]
        gcnt = jnp.int32(0)
        fcbs = []
        for b in range(16):
            fcb = jnp.cumsum(blocks[b]) + pre_before[b]
            fcbs.append(fcb)
            gcnt = gcnt + jnp.sum((fcb < kk).astype(jnp.int32))
        prev = jnp.int32(0)
        for b in range(16):
            lid = iota16 + 16 * b
            prev = prev + jnp.sum(jnp.where(lid == gcnt - 1, fcbs[b], 0))
        return gcnt, prev

    def val_iter(t, carry):
        vp, kp1, vn, kn1, _, _ = carry
        sh = jnp.uint32(24) - jnp.uint32(8) * t.astype(jnp.uint32)
        reg = (t & 1) * 8192
        zero_hist()

        def chunk(c, _):
            xp = kup[pl.ds(c * 16, 16)]
            xn = kun[pl.ds(c * 16, 16)]
            vmp = (xp >> sh >> 8) == (vp >> sh >> 8)
            vmn = (xn >> sh >> 8) == (vn >> sh >> 8)
            nibp = ((xp >> sh) & jnp.uint32(255)).astype(jnp.int32)
            nibn = ((xn >> sh) & jnp.uint32(255)).astype(jnp.int32) + 256
            plsc.addupdate_scatter(hist, [nibp], ones16, mask=vmp)
            plsc.addupdate_scatter(hist, [nibn], ones16, mask=vmn)
            return 0

        lax.fori_loop(0, _NCH, chunk, 0, unroll=8)
        blocks = exchange(reg)
        gp, nxtp, tiep = pick_desc(blocks[:16], kp1)
        gn, nxtn, tien = pick_desc(blocks[16:], kn1)
        kp1 = kp1 - nxtp
        kn1 = kn1 - nxtn
        vp = vp | (gp.astype(jnp.uint32) << sh)
        vn = vn | (gn.astype(jnp.uint32) << sh)
        return vp, kp1, vn, kn1, tiep, tien

    vp, kp1, vn, kn1, tiecp, tiecn = lax.fori_loop(
        0, 4, val_iter,
        (jnp.uint32(0), k, jnp.uint32(0), k, jnp.int32(0), jnp.int32(0)))

    skip = (tiecp == kp1) & (tiecn == kn1)
    trip = jnp.where(skip, 0, 2)
    init_ip = jnp.where(skip, imax, jnp.int32(0))

    def idx_iter(t, carry):
        ipx, k2p, inx, k2n = carry
        sh = 8 - 8 * t
        reg = (t & 1) * 8192
        zero_hist()

        def chunk(c, _):
            xp = kup[pl.ds(c * 16, 16)]
            xn = kun[pl.ds(c * 16, 16)]
            gidx = base + c * 16 + iota16
            vmp = (xp == vp) & ((gidx >> sh >> 8) == (ipx >> sh >> 8))
            vmn = (xn == vn) & ((gidx >> sh >> 8) == (inx >> sh >> 8))
            nib = (gidx >> sh) & 255
            plsc.addupdate_scatter(hist, [nib], ones16, mask=vmp)
            plsc.addupdate_scatter(hist, [nib + 256], ones16, mask=vmn)
            return 0

        lax.fori_loop(0, _NCH, chunk, 0, unroll=8)
        blocks = exchange(reg)
        gp, prevp = pick_asc(blocks[:16], k2p)
        gn, prevn = pick_asc(blocks[16:], k2n)
        k2p = k2p - prevp
        k2n = k2n - prevn
        ipx = ipx | (gp << sh)
        inx = inx | (gn << sh)
        return ipx, k2p, inx, k2n

    tpx, _, tnx, _ = lax.fori_loop(
        0, trip, idx_iter, (init_ip, kp1, init_ip, kn1))

    def psum(c, acc):
        xp = kup[pl.ds(c * 16, 16)]
        xn = kun[pl.ds(c * 16, 16)]
        gidx = base + c * 16 + iota16
        sp = (xp > vp) | ((xp == vp) & (gidx <= tpx))
        sn = (xn > vn) | ((xn == vn) & (gidx <= tnx))
        v = jnp.where(sp, cfv[pl.ds(c * 16, 16)], 0.0)
        v = v + jnp.where(sn, cbv[pl.ds(c * 16, 16)], 0.0)
        return acc + v

    part = lax.fori_loop(0, _NCH, psum, jnp.zeros((16,), jnp.float32), unroll=8)
    stage[...] = part
    pltpu.sync_copy(stage, sharedf.at[pl.ds(wid * 16, 16)])
    plsc.subcore_barrier()

    @pl.when((cid == 0) & (wid == 0))
    def _():
        pltpu.sync_copy(sharedf, allf)

        def fs(w, acc):
            return acc + allf[pl.ds(w * 16, 16)]

        tot = lax.fori_loop(0, _NSUB, fs, jnp.zeros((16,), jnp.float32))
        stage[...] = (jnp.broadcast_to(jnp.sum(tot), (16,))
                      * jnp.float32(-1.0 / (2 * _K)))
        pltpu.sync_copy(stage, out_hbm)


def _make_sc_select():
    mesh = plsc.VectorSubcoreMesh(core_axis_name="c", subcore_axis_name="s")
    return pl.kernel(
        _sc_select_body,
        out_type=jax.ShapeDtypeStruct((16,), jnp.float32),
        mesh=mesh,
        compiler_params=pltpu.CompilerParams(needs_layout_passes=False),
        scratch_types=[
            pltpu.VMEM((_PER,), jnp.uint32),
            pltpu.VMEM((_PER,), jnp.uint32),
            pltpu.VMEM((_PER,), jnp.float32),
            pltpu.VMEM((_PER,), jnp.float32),
            pltpu.VMEM((512,), jnp.int32),
            pltpu.VMEM((8192,), jnp.int32),
            pltpu.VMEM((256,), jnp.float32),
            pltpu.VMEM((16,), jnp.float32),
            pltpu.VMEM_SHARED((16384,), jnp.int32),
            pltpu.VMEM_SHARED((256,), jnp.float32),
        ],
    )


@jax.jit
def kernel(scores, labels, un_id, weight, bias):
    st = jnp.pad(scores.T, ((0, 0), (0, _NP - _N)))
    lab2 = jnp.pad(labels.reshape(1, _N), ((0, 0), (0, _NP - _N)))
    pos, neg, cfg, cbg = pl.pallas_call(
        _rowstats_kernel,
        grid=(_GRID,),
        in_specs=[
            pl.BlockSpec((_NC + 1, _CHUNK), lambda i: (0, i)),
            pl.BlockSpec((1, _CHUNK), lambda i: (0, i)),
        ],
        out_specs=[pl.BlockSpec((1, _CHUNK), lambda i: (0, i))] * 4,
        out_shape=[jax.ShapeDtypeStruct((1, _NP), jnp.int32)] * 2
        + [jax.ShapeDtypeStruct((1, _NP), jnp.float32)] * 2,
    )(st, lab2)

    kpu = lax.bitcast_convert_type(pos.reshape(_NP), jnp.uint32)
    knu = lax.bitcast_convert_type(neg.reshape(_NP), jnp.uint32)
    out = _make_sc_select()(kpu, knu, cfg.reshape(_NP), cbg.reshape(_NP))
    return out[0]


# speedup vs baseline: 1.1458x; 1.0183x over previous
import jax
import jax.numpy as jnp
from jax import lax
from jax.experimental import pallas as pl
from jax.experimental.pallas import tpu as pltpu
from jax.experimental.pallas import tpu_sc as plsc

_NC = 81
_N = 20000
_K = 64
_NP = 20480
_CHUNK = 10240
_GRID = _NP // _CHUNK
_ROWS = _NP // 128
_MIN32 = -2147483648


def _rowstats_kernel(st_ref, lab_ref, pos_ref, neg_ref, cfg_ref, cbg_ref):
    s = st_ref[...]
    lab = lab_ref[...]
    gcol = (pl.program_id(0) * _CHUNK
            + lax.broadcasted_iota(jnp.int32, (1, _CHUNK), 1))
    valid = gcol < _N

    m80 = jnp.max(s[:_NC - 1, :], axis=0, keepdims=True)
    m_ex = jnp.maximum(m80, s[_NC:_NC + 1, :])
    metric = -m_ex
    b = lax.bitcast_convert_type(metric, jnp.int32)
    key = jnp.where(b < 0, b ^ jnp.int32(0x7FFFFFFF), b)
    fg = (lab != _NC) & valid
    bg = (lab == _NC) & valid
    masked = jnp.int32(_MIN32)
    pos_ref[...] = jnp.where(fg, key, masked)
    neg_ref[...] = jnp.where(bg, key, masked)

    m_all = jnp.maximum(m_ex, s[_NC - 1:_NC, :])
    e = jnp.exp(s - m_all)
    ssum = jnp.sum(e, axis=0, keepdims=True)
    row = lax.broadcasted_iota(jnp.int32, s.shape, 0)
    e_lab = jnp.sum(jnp.where(row == lab, e, 0.0), axis=0, keepdims=True)
    gt = e_lab / ssum
    t = gt * (1.0 - gt)
    denomlog = m_all + jnp.log(ssum - e_lab)
    s79 = s[_NC - 2:_NC - 1, :]
    s80 = s[_NC - 1:_NC, :]
    s81 = s[_NC:_NC + 1, :]
    c_fg = jnp.where(lab <= _NC - 2, s80, s79)
    c_bg = jnp.where(lab <= _NC - 1, s81, s80)
    cfg_ref[...] = t * (c_fg - denomlog)
    cbg_ref[...] = t * (c_bg - denomlog)


_NSUB = 16
_PER = _NP // _NSUB
_NCH = _PER // 16


def _sc_select_body(kpu_hbm, knu_hbm, cf_hbm, cb_hbm, out_hbm,
                    kup, kun, cfv, cbv, hist, allh, allf, stage,
                    sharedh, sharedf):
    cid = lax.axis_index("c")
    wid = lax.axis_index("s")
    base = wid * _PER
    iota16 = lax.iota(jnp.int32, 16)
    ones16 = jnp.ones((16,), jnp.int32)
    zeros16 = jnp.zeros((16,), jnp.int32)
    k = jnp.int32(_K)
    imax = jnp.int32(0x7FFFFFFF)

    pltpu.sync_copy(kpu_hbm.at[pl.ds(base, _PER)], kup)
    pltpu.sync_copy(knu_hbm.at[pl.ds(base, _PER)], kun)
    pltpu.sync_copy(cf_hbm.at[pl.ds(base, _PER)], cfv)
    pltpu.sync_copy(cb_hbm.at[pl.ds(base, _PER)], cbv)

    sgn = jnp.uint32(0x80000000)

    def conv(c, _):
        kup[pl.ds(c * 16, 16)] = kup[pl.ds(c * 16, 16)] ^ sgn
        kun[pl.ds(c * 16, 16)] = kun[pl.ds(c * 16, 16)] ^ sgn
        return 0

    lax.fori_loop(0, _NCH, conv, 0, unroll=8)

    def zero_hist():
        for i in range(32):
            hist[pl.ds(i * 16, 16)] = zeros16

    def exchange(reg):
        pltpu.sync_copy(hist, sharedh.at[pl.ds(reg + wid * 512, 512)])
        plsc.subcore_barrier()
        pltpu.sync_copy(sharedh.at[pl.ds(reg, 8192)], allh)

        def srow(w, acc):
            return tuple(acc[b] + allh[pl.ds(w * 512 + b * 16, 16)]
                         for b in range(32))

        return lax.fori_loop(0, _NSUB, srow, (zeros16,) * 32)

    def pick_desc(blocks, kk):
        bs = [jnp.sum(blocks[b]) for b in range(16)]
        suf = jnp.int32(0)
        suf_after = [None] * 16
        for b in range(15, -1, -1):
            suf_after[b] = suf
            suf = suf + bs[b]
        gcnt = jnp.int32(0)
        rcbs = []
        for b in range(16):
            rcb = jnp.flip(jnp.cumsum(jnp.flip(blocks[b], 0)), 0) + suf_after[b]
            rcbs.append(rcb)
            gcnt = gcnt + jnp.sum((rcb >= kk).astype(jnp.int32))
        g = gcnt - 1
        rc_g = jnp.int32(0)
        rc_g1 = jnp.int32(0)
        for b in range(16):
            lid = iota16 + 16 * b
            rc_g = rc_g + jnp.sum(jnp.where(lid == g, rcbs[b], 0))
            rc_g1 = rc_g1 + jnp.sum(jnp.where(lid == g + 1, rcbs[b], 0))
        return g, rc_g1, rc_g - rc_g1

    def pick_asc(blocks, kk):
        bs = [jnp.sum(blocks[b]) for b in range(16)]
        pre = jnp.int32(0)
        pre_before = []
        for b in range(16):
            pre_before.append(pre)
            pre = pre + bs[b]
        gcnt = jnp.int32(0)
        fcbs = []
        for b in range(16):
            fcb = jnp.cumsum(blocks[b]) + pre_before[b]
            fcbs.append(fcb)
            gcnt = gcnt + jnp.sum((fcb < kk).astype(jnp.int32))
        prev = jnp.int32(0)
        for b in range(16):
            lid = iota16 + 16 * b
            prev = prev + jnp.sum(jnp.where(lid == gcnt - 1, fcbs[b], 0))
        return gcnt, prev

    def val_iter(t, carry):
        vp, kp1, vn, kn1, _, _ = carry
        sh = jnp.uint32(24) - jnp.uint32(8) * t.astype(jnp.uint32)
        reg = (t & 1) * 8192
        zero_hist()

        def chunk(c, _):
            xp = kup[pl.ds(c * 16, 16)]
            xn = kun[pl.ds(c * 16, 16)]
            vmp = (xp >> sh >> 8) == (vp >> sh >> 8)
            vmn = (xn >> sh >> 8) == (vn >> sh >> 8)
            nibp = ((xp >> sh) & jnp.uint32(255)).astype(jnp.int32)
            nibn = ((xn >> sh) & jnp.uint32(255)).astype(jnp.int32) + 256
            plsc.addupdate_scatter(hist, [nibp], ones16, mask=vmp)
            plsc.addupdate_scatter(hist, [nibn], ones16, mask=vmn)
            return 0

        lax.fori_loop(0, _NCH, chunk, 0, unroll=8)
        blocks = exchange(reg)
        gp, nxtp, tiep = pick_desc(blocks[:16], kp1)
        gn, nxtn, tien = pick_desc(blocks[16:], kn1)
        kp1 = kp1 - nxtp
        kn1 = kn1 - nxtn
        vp = vp | (gp.astype(jnp.uint32) << sh)
        vn = vn | (gn.astype(jnp.uint32) << sh)
        return vp, kp1, vn, kn1, tiep, tien

    vp, kp1, vn, kn1, tiecp, tiecn = lax.fori_loop(
        0, 4, val_iter,
        (jnp.uint32(0), k, jnp.uint32(0), k, jnp.int32(0), jnp.int32(0)))

    skip = (tiecp == kp1) & (tiecn == kn1)
    trip = jnp.where(skip, 0, 2)
    init_ip = jnp.where(skip, imax, jnp.int32(0))

    def idx_iter(t, carry):
        ipx, k2p, inx, k2n = carry
        sh = 8 - 8 * t
        reg = (t & 1) * 8192
        zero_hist()

        def chunk(c, _):
            xp = kup[pl.ds(c * 16, 16)]
            xn = kun[pl.ds(c * 16, 16)]
            gidx = base + c * 16 + iota16
            vmp = (xp == vp) & ((gidx >> sh >> 8) == (ipx >> sh >> 8))
            vmn = (xn == vn) & ((gidx >> sh >> 8) == (inx >> sh >> 8))
            nib = (gidx >> sh) & 255
            plsc.addupdate_scatter(hist, [nib], ones16, mask=vmp)
            plsc.addupdate_scatter(hist, [nib + 256], ones16, mask=vmn)
            return 0

        lax.fori_loop(0, _NCH, chunk, 0, unroll=8)
        blocks = exchange(reg)
        gp, prevp = pick_asc(blocks[:16], k2p)
        gn, prevn = pick_asc(blocks[16:], k2n)
        k2p = k2p - prevp
        k2n = k2n - prevn
        ipx = ipx | (gp << sh)
        inx = inx | (gn << sh)
        return ipx, k2p, inx, k2n

    tpx, _, tnx, _ = lax.fori_loop(
        0, trip, idx_iter, (init_ip, kp1, init_ip, kn1))

    def psum(c, acc):
        xp = kup[pl.ds(c * 16, 16)]
        xn = kun[pl.ds(c * 16, 16)]
        gidx = base + c * 16 + iota16
        sp = (xp > vp) | ((xp == vp) & (gidx <= tpx))
        sn = (xn > vn) | ((xn == vn) & (gidx <= tnx))
        v = jnp.where(sp, cfv[pl.ds(c * 16, 16)], 0.0)
        v = v + jnp.where(sn, cbv[pl.ds(c * 16, 16)], 0.0)
        return acc + v

    part = lax.fori_loop(0, _NCH, psum, jnp.zeros((16,), jnp.float32), unroll=8)
    stage[...] = part
    pltpu.sync_copy(stage, sharedf.at[pl.ds(wid * 16, 16)])
    plsc.subcore_barrier()

    @pl.when((cid == 0) & (wid == 0))
    def _():
        pltpu.sync_copy(sharedf, allf)

        def fs(w, acc):
            return acc + allf[pl.ds(w * 16, 16)]

        tot = lax.fori_loop(0, _NSUB, fs, jnp.zeros((16,), jnp.float32))
        stage[...] = (jnp.broadcast_to(jnp.sum(tot), (16,))
                      * jnp.float32(-1.0 / (2 * _K)))
        pltpu.sync_copy(stage, out_hbm)


def _make_sc_select():
    mesh = plsc.VectorSubcoreMesh(core_axis_name="c", subcore_axis_name="s")
    return pl.kernel(
        _sc_select_body,
        out_type=jax.ShapeDtypeStruct((16,), jnp.float32),
        mesh=mesh,
        compiler_params=pltpu.CompilerParams(needs_layout_passes=False),
        scratch_types=[
            pltpu.VMEM((_PER,), jnp.uint32),
            pltpu.VMEM((_PER,), jnp.uint32),
            pltpu.VMEM((_PER,), jnp.float32),
            pltpu.VMEM((_PER,), jnp.float32),
            pltpu.VMEM((512,), jnp.int32),
            pltpu.VMEM((8192,), jnp.int32),
            pltpu.VMEM((256,), jnp.float32),
            pltpu.VMEM((16,), jnp.float32),
            pltpu.VMEM_SHARED((16384,), jnp.int32),
            pltpu.VMEM_SHARED((256,), jnp.float32),
        ],
    )


@jax.jit
def kernel(scores, labels, un_id, weight, bias):
    st = jnp.pad(scores.T, ((0, 0), (0, _NP - _N)))
    lab2 = jnp.pad(labels.reshape(1, _N), ((0, 0), (0, _NP - _N)))
    pos, neg, cfg, cbg = pl.pallas_call(
        _rowstats_kernel,
        grid=(_GRID,),
        in_specs=[
            pl.BlockSpec((_NC + 1, _CHUNK), lambda i: (0, i)),
            pl.BlockSpec((1, _CHUNK), lambda i: (0, i)),
        ],
        out_specs=[pl.BlockSpec((1, _CHUNK), lambda i: (0, i))] * 4,
        out_shape=[jax.ShapeDtypeStruct((1, _NP), jnp.int32)] * 2
        + [jax.ShapeDtypeStruct((1, _NP), jnp.float32)] * 2,
    )(st, lab2)

    kpu = lax.bitcast_convert_type(pos.reshape(_NP), jnp.uint32)
    knu = lax.bitcast_convert_type(neg.reshape(_NP), jnp.uint32)
    out = _make_sc_select()(kpu, knu, cfg.reshape(_NP), cbg.reshape(_NP))
    return out[0]
